# baseline (speedup 1.0000x reference)
import functools

import jax
import jax.numpy as jnp
from jax import lax
from jax.experimental import pallas as pl
from jax.experimental.pallas import tpu as pltpu
from jax.experimental.pallas import tpu_sc as plsc

_NC, _NS, _L = 2, 16, 16
_NW = _NC * _NS
_B, _F = 16384, 26
_R = _B // _NW
_K = _R * _F
_CH = 128
_NCHUNK = _K // _CH


def _body(idx_hbm, w_hbm, out_hbm, idx_v, vals_v, out_v, sem):
    wid = lax.axis_index("s") * _NC + lax.axis_index("c")
    base = pl.multiple_of(wid * _K, _K)
    pltpu.sync_copy(idx_hbm.at[pl.ds(base, _K)], idx_v)

    pltpu.async_copy(w_hbm.at[idx_v], vals_v, sem).wait()

    lanes = lax.iota(jnp.int32, _L) * _F

    def red(j, c):
        p0 = lanes + j * (_L * _F)
        acc = plsc.load_gather(vals_v, [p0])
        for f in range(1, _F):
            acc = acc + plsc.load_gather(vals_v, [p0 + f])
        out_v[pl.ds(pl.multiple_of(j * _L, _L), _L)] = acc
        return c

    lax.fori_loop(0, _R // _L, red, 0)
    pltpu.sync_copy(out_v, out_hbm.at[pl.ds(pl.multiple_of(wid * _R, _R), _R)])


_sc_call = pl.kernel(
    _body,
    out_type=jax.ShapeDtypeStruct((_B,), jnp.float32),
    mesh=plsc.VectorSubcoreMesh(
        core_axis_name="c", subcore_axis_name="s",
        num_cores=_NC, num_subcores=_NS,
    ),
    scratch_types=[
        pltpu.VMEM((_K,), jnp.int32),
        pltpu.VMEM((_K,), jnp.float32),
        pltpu.VMEM((_R,), jnp.float32),
        pltpu.SemaphoreType.DMA,
    ],
    compiler_params=pltpu.CompilerParams(needs_layout_passes=False),
)


@jax.jit
def kernel(inputs, w):
    idx = inputs.astype(jnp.int32).reshape(-1)
    table = w.reshape(-1)
    return _sc_call(idx, table).reshape(_B, 1)


# speedup vs baseline: 1.0014x; 1.0014x over previous
import functools

import jax
import jax.numpy as jnp
from jax import lax
from jax.experimental import pallas as pl
from jax.experimental.pallas import tpu as pltpu
from jax.experimental.pallas import tpu_sc as plsc

_NC, _NS, _L = 2, 16, 16
_NW = _NC * _NS
_B, _F = 16384, 26
_R = _B // _NW
_K = _R * _F
_CH = 128
_NCHUNK = _K // _CH


def _body(idx_hbm, w_hbm, out_hbm, idx_v, vals_v, out_v, sem):
    wid = lax.axis_index("s") * _NC + lax.axis_index("c")
    base = pl.multiple_of(wid * _K, _K)
    pltpu.sync_copy(idx_hbm.at[pl.ds(base, _K)], idx_v)

    pltpu.async_copy(w_hbm.at[idx_v], vals_v, sem).wait()

    lanes = lax.iota(jnp.int32, _L) * _F

    def red(j, c):
        p0 = lanes + j * (_L * _F)
        acc = plsc.load_gather(vals_v, [p0])
        for f in range(1, _F):
            acc = acc + plsc.load_gather(vals_v, [p0 + f])
        out_v[pl.ds(pl.multiple_of(j * _L, _L), _L)] = acc
        return c

    lax.fori_loop(0, _R // _L, red, 0)
    pltpu.sync_copy(out_v, out_hbm.at[pl.ds(pl.multiple_of(wid * _R, _R), _R)])


_sc_call = pl.kernel(
    _body,
    out_type=jax.ShapeDtypeStruct((_B,), jnp.float32),
    mesh=plsc.VectorSubcoreMesh(
        core_axis_name="c", subcore_axis_name="s",
        num_cores=_NC, num_subcores=_NS,
    ),
    scratch_types=[
        pltpu.VMEM((_K,), jnp.int32),
        pltpu.VMEM((_K,), jnp.float32),
        pltpu.VMEM((_R,), jnp.float32),
        pltpu.SemaphoreType.DMA,
    ],
    compiler_params=pltpu.CompilerParams(needs_layout_passes=False),
)


@jax.jit
def kernel(inputs, w):
    idx = inputs.astype(jnp.int32).reshape(-1)
    return _sc_call(idx, w[:, 0]).reshape(_B, 1)


# speedup vs baseline: 1.0301x; 1.0287x over previous
import functools

import jax
import jax.numpy as jnp
from jax import lax
from jax.experimental import pallas as pl
from jax.experimental.pallas import tpu as pltpu
from jax.experimental.pallas import tpu_sc as plsc

_NC, _NS, _L = 2, 16, 16
_NW = _NC * _NS
_B, _F = 16384, 26
_R = _B // _NW


def _body(idxT_hbm, w_hbm, out_hbm, *s):
    idxs = s[0:_F]
    vals = s[_F:2 * _F]
    out_v = s[2 * _F]
    sem = s[2 * _F + 1]
    wid = lax.axis_index("s") * _NC + lax.axis_index("c")
    col0 = pl.multiple_of(wid * _R, _R)
    for f in range(_F):
        pltpu.sync_copy(idxT_hbm.at[f, pl.ds(col0, _R)], idxs[f])

    cps = [
        pltpu.async_copy(w_hbm.at[idxs[f]], vals[f], sem)
        for f in range(_F)
    ]
    for c in cps:
        c.wait()

    def red(j, c):
        base = pl.multiple_of(j * _L, _L)
        acc = vals[0][pl.ds(base, _L)]
        for f in range(1, _F):
            acc = acc + vals[f][pl.ds(base, _L)]
        out_v[pl.ds(base, _L)] = acc
        return c

    lax.fori_loop(0, _R // _L, red, 0)
    pltpu.sync_copy(out_v, out_hbm.at[pl.ds(col0, _R)])


_sc_call = pl.kernel(
    _body,
    out_type=jax.ShapeDtypeStruct((_B,), jnp.float32),
    mesh=plsc.VectorSubcoreMesh(
        core_axis_name="c", subcore_axis_name="s",
        num_cores=_NC, num_subcores=_NS,
    ),
    scratch_types=(
        [pltpu.VMEM((_R,), jnp.int32) for _ in range(_F)]
        + [pltpu.VMEM((_R,), jnp.float32) for _ in range(_F)]
        + [pltpu.VMEM((_R,), jnp.float32), pltpu.SemaphoreType.DMA]
    ),
    compiler_params=pltpu.CompilerParams(needs_layout_passes=False),
)


@jax.jit
def kernel(inputs, w):
    idxT = inputs.astype(jnp.int32).T
    return _sc_call(idxT, w.reshape(-1)).reshape(_B, 1)


# speedup vs baseline: 1.2034x; 1.1683x over previous
import functools

import jax
import jax.numpy as jnp
from jax import lax
from jax.experimental import pallas as pl
from jax.experimental.pallas import tpu as pltpu
from jax.experimental.pallas import tpu_sc as plsc

_NC, _NS, _L = 2, 16, 16
_NW = _NC * _NS
_B, _F = 16384, 26
_R = _B // _NW


def _body(idxT_hbm, w_hbm, out_hbm, *s):
    idxs = s[0:_F]
    vals = s[_F:2 * _F]
    out_v = s[2 * _F]
    sem = s[2 * _F + 1]
    sem_idx = s[2 * _F + 2]
    wid = lax.axis_index("s") * _NC + lax.axis_index("c")
    col0 = pl.multiple_of(wid * _R, _R)
    icps = [
        pltpu.async_copy(idxT_hbm.at[f, pl.ds(col0, _R)], idxs[f], sem_idx)
        for f in range(_F)
    ]
    gcps = []
    for f in range(_F):
        icps[f].wait()
        gcps.append(pltpu.async_copy(w_hbm.at[idxs[f]], vals[f], sem))
    for c in gcps:
        c.wait()

    def red(j, c):
        base = pl.multiple_of(j * _L, _L)
        acc = vals[0][pl.ds(base, _L)]
        for f in range(1, _F):
            acc = acc + vals[f][pl.ds(base, _L)]
        out_v[pl.ds(base, _L)] = acc
        return c

    lax.fori_loop(0, _R // _L, red, 0)
    pltpu.sync_copy(out_v, out_hbm.at[pl.ds(col0, _R)])


_sc_call = pl.kernel(
    _body,
    out_type=jax.ShapeDtypeStruct((_B,), jnp.float32),
    mesh=plsc.VectorSubcoreMesh(
        core_axis_name="c", subcore_axis_name="s",
        num_cores=_NC, num_subcores=_NS,
    ),
    scratch_types=(
        [pltpu.VMEM((_R,), jnp.int32) for _ in range(_F)]
        + [pltpu.VMEM((_R,), jnp.float32) for _ in range(_F)]
        + [pltpu.VMEM((_R,), jnp.float32),
           pltpu.SemaphoreType.DMA, pltpu.SemaphoreType.DMA]
    ),
    compiler_params=pltpu.CompilerParams(needs_layout_passes=False),
)


@jax.jit
def kernel(inputs, w):
    idxT = inputs.astype(jnp.int32).T
    return _sc_call(idxT, w.reshape(-1)).reshape(_B, 1)
